# baseline (speedup 1.0000x reference)
import functools

import jax
import jax.numpy as jnp
from jax import lax
from jax.experimental import pallas as pl
from jax.experimental.pallas import tpu as pltpu
from jax.experimental.pallas import tpu_sc as plsc

_NUM_CORES = 2
_NUM_SUBCORES = 16
_K = 128


def _edge_agg_sc(x, edge_index, zeros):
    N, D = x.shape
    E = edge_index.shape[1]
    NW = _NUM_CORES * _NUM_SUBCORES
    chunks = E // _K
    base_cnt = chunks // NW
    extra = chunks - base_cnt * NW
    slots = base_cnt + (1 if extra else 0)
    unroll = 12
    iters = (slots + 2 + unroll) // unroll
    nr = (N // _NUM_SUBCORES) // 8 * 8
    tail = N - nr * _NUM_SUBCORES
    mesh = plsc.VectorSubcoreMesh(core_axis_name="c", subcore_axis_name="s")

    @functools.partial(
        pl.kernel,
        out_type=jax.ShapeDtypeStruct((_NUM_CORES, N, D), jnp.float32),
        mesh=mesh,
        scratch_types=[
            pltpu.VMEM((4, _K), jnp.int32),
            pltpu.VMEM((4, _K), jnp.int32),
            pltpu.VMEM((_K, D), jnp.float32),
            pltpu.VMEM((_K, D), jnp.float32),
            pltpu.VMEM((_K, D), jnp.float32),
            pltpu.VMEM_SHARED((N, D), jnp.float32),
        ] + [pltpu.SemaphoreType.DMA] * 15,
    )
    def k(x_hbm, ei_hbm, z_hbm, out_hbm,
          srcb, dstb, rows0, rows1, rows2, agg_sh,
          sg0, sg1, sg2, ss0, ss1, ss2,
          sa0, sa1, sa2, sa3, sd0, sd1, sd2, sd3, sz):
        c = lax.axis_index("c")
        s = lax.axis_index("s")
        wid = c * _NUM_SUBCORES + s
        start = wid * base_cnt + jnp.minimum(wid, extra)
        cnt = base_cnt + jnp.where(wid < extra, 1, 0)
        rows = (rows0, rows1, rows2)
        sg = (sg0, sg1, sg2)
        ss = (ss0, ss1, ss2)
        sa = (sa0, sa1, sa2, sa3)
        sd = (sd0, sd1, sd2, sd3)

        def src_copy(slot, r):
            off = (start + slot) * _K
            return pltpu.make_async_copy(
                ei_hbm.at[0, pl.ds(off, _K)], srcb.at[r], sa[r]
            )

        def dst_copy(slot, r):
            off = (start + slot) * _K
            return pltpu.make_async_copy(
                ei_hbm.at[1, pl.ds(off, _K)], dstb.at[r], sd[r]
            )

        def gather_copy(r, par):
            return pltpu.make_async_copy(
                x_hbm.at[srcb.at[r]], rows[par], sg[par]
            )

        def scat_copy(r, par):
            return pltpu.make_async_copy(
                rows[par], agg_sh.at[dstb.at[r]], ss[par]
            )

        zero_cp = pltpu.make_async_copy(
            z_hbm.at[pl.ds(0, nr)], agg_sh.at[pl.ds(s * nr, nr)], sz
        )
        zero_cp.start()
        if tail:
            @pl.when(s == _NUM_SUBCORES - 1)
            def _():
                pltpu.async_copy(
                    z_hbm.at[pl.ds(0, tail)],
                    agg_sh.at[pl.ds(nr * _NUM_SUBCORES, tail)],
                    sz,
                )

        src_copy(0, 0).start()
        src_copy(1, 1).start()
        src_copy(2, 2).start()
        dst_copy(0, 0).start()
        dst_copy(1, 1).start()
        src_copy(0, 0).wait()
        gather_copy(0, 0).start()
        zero_cp.wait()
        if tail:
            @pl.when(s == _NUM_SUBCORES - 1)
            def _():
                pltpu.make_async_copy(
                    z_hbm.at[pl.ds(0, tail)],
                    agg_sh.at[pl.ds(nr * _NUM_SUBCORES, tail)],
                    sz,
                ).wait()
        plsc.subcore_barrier()

        @pl.loop(0, iters)
        def _(q):
            for j in range(unroll):
                i = unroll * q + j

                @pl.when(jnp.logical_and(i >= 2, i - 2 < cnt))
                def _():
                    scat_copy((j + 2) % 4, (j + 1) % 3).wait()

                @pl.when(i + 1 < cnt)
                def _():
                    src_copy(i + 1, (j + 1) % 4).wait()
                    gather_copy((j + 1) % 4, (j + 1) % 3).start()

                @pl.when(i < cnt)
                def _():
                    gather_copy(j % 4, j % 3).wait()
                    dst_copy(i, j % 4).wait()
                    scat_copy(j % 4, j % 3).start(add=True)

                @pl.when(i + 3 < cnt)
                def _():
                    src_copy(i + 3, (j + 3) % 4).start()

                @pl.when(i + 2 < cnt)
                def _():
                    dst_copy(i + 2, (j + 2) % 4).start()

        plsc.subcore_barrier()
        pltpu.sync_copy(
            agg_sh.at[pl.ds(s * nr, nr)], out_hbm.at[c, pl.ds(s * nr, nr)]
        )
        if tail:
            @pl.when(s == _NUM_SUBCORES - 1)
            def _():
                pltpu.sync_copy(
                    agg_sh.at[pl.ds(nr * _NUM_SUBCORES, tail)],
                    out_hbm.at[c, pl.ds(nr * _NUM_SUBCORES, tail)],
                )

    return k(x, edge_index, zeros)


def _x_self(x, W_self, b_row):
    N, D = x.shape

    def body(x_ref, ws_ref, b_ref, o_ref):
        o_ref[...] = (
            jnp.dot(x_ref[...], ws_ref[...], preferred_element_type=jnp.float32)
            + b_ref[...]
        )

    return pl.pallas_call(
        body, out_shape=jax.ShapeDtypeStruct((N, D), jnp.float32)
    )(x, W_self, b_row)


def _post(agg2, x_self, W_msg, att_w_row, batch_row, G):
    N, D = x_self.shape

    def body(agg_ref, xs_ref, wm_ref, aw_ref, bat_ref, out_ref):
        aggm = jnp.dot(
            agg_ref[0] + agg_ref[1], wm_ref[...],
            preferred_element_type=jnp.float32,
            precision=lax.Precision.HIGHEST,
        )
        h = jnp.maximum(aggm + xs_ref[...], 0.0)
        logits = lax.dot_general(
            aw_ref[...], h, (((1,), (1,)), ((), ())),
            preferred_element_type=jnp.float32,
        )
        bat = bat_ref[...]
        gid = lax.broadcasted_iota(jnp.int32, (G, N), 0)
        mask = gid == bat
        big_neg = jnp.float32(-1e30)
        seg_max = jnp.max(jnp.where(mask, logits, big_neg), axis=1, keepdims=True)
        maxn = jnp.sum(jnp.where(mask, seg_max, 0.0), axis=0, keepdims=True)
        w = jnp.exp(logits - maxn)
        denom = jnp.sum(jnp.where(mask, w, 0.0), axis=1, keepdims=True)
        denn = jnp.sum(jnp.where(mask, denom, 0.0), axis=0, keepdims=True)
        wn = w / denn
        out_ref[...] = jnp.dot(
            jnp.where(mask, wn, 0.0), h, preferred_element_type=jnp.float32
        )

    return pl.pallas_call(
        body, out_shape=jax.ShapeDtypeStruct((G, D), jnp.float32)
    )(agg2, x_self, W_msg, att_w_row, batch_row)


def kernel(x, edge_index, batch, W_msg, W_self, b, att_w, att_b):
    N, D = x.shape
    G = 64
    zeros = jnp.zeros((640, D), jnp.float32)
    agg2 = _edge_agg_sc(x, edge_index, zeros)
    x_self = _x_self(x, W_self, b.reshape(1, D))
    out = _post(agg2, x_self, W_msg, att_w.reshape(1, D),
                batch.reshape(1, N), G)
    return out


# speedup vs baseline: 1.0296x; 1.0296x over previous
import functools

import jax
import jax.numpy as jnp
from jax import lax
from jax.experimental import pallas as pl
from jax.experimental.pallas import tpu as pltpu
from jax.experimental.pallas import tpu_sc as plsc

_NUM_CORES = 2
_NUM_SUBCORES = 16
_K = 128


def _edge_agg_sc(x, edge_index, zeros):
    N, D = x.shape
    E = edge_index.shape[1]
    NW = _NUM_CORES * _NUM_SUBCORES
    chunks = E // _K
    base_cnt = chunks // NW
    extra = chunks - base_cnt * NW
    slots = base_cnt + (1 if extra else 0)
    unroll = 12
    iters = (slots + 2 + unroll) // unroll
    nr = (N // _NUM_SUBCORES) // 8 * 8
    tail = N - nr * _NUM_SUBCORES
    mesh = plsc.VectorSubcoreMesh(core_axis_name="c", subcore_axis_name="s")

    @functools.partial(
        pl.kernel,
        out_type=jax.ShapeDtypeStruct((_NUM_CORES, N, D), jnp.float32),
        mesh=mesh,
        scratch_types=[
            pltpu.VMEM((4, _K), jnp.int32),
            pltpu.VMEM((4, _K), jnp.int32),
            pltpu.VMEM((_K, D), jnp.float32),
            pltpu.VMEM((_K, D), jnp.float32),
            pltpu.VMEM((_K, D), jnp.float32),
            pltpu.VMEM_SHARED((N, D), jnp.float32),
        ] + [pltpu.SemaphoreType.DMA] * 15,
    )
    def k(x_hbm, ei_hbm, z_hbm, out_hbm,
          srcb, dstb, rows0, rows1, rows2, agg_sh,
          sg0, sg1, sg2, ss0, ss1, ss2,
          sa0, sa1, sa2, sa3, sd0, sd1, sd2, sd3, sz):
        c = lax.axis_index("c")
        s = lax.axis_index("s")
        wid = c * _NUM_SUBCORES + s
        start = wid * base_cnt + jnp.minimum(wid, extra)
        cnt = base_cnt + jnp.where(wid < extra, 1, 0)
        rows = (rows0, rows1, rows2)
        sg = (sg0, sg1, sg2)
        ss = (ss0, ss1, ss2)
        sa = (sa0, sa1, sa2, sa3)
        sd = (sd0, sd1, sd2, sd3)

        def src_copy(slot, r):
            off = (start + slot) * _K
            return pltpu.make_async_copy(
                ei_hbm.at[0, pl.ds(off, _K)], srcb.at[r], sa[r]
            )

        def dst_copy(slot, r):
            off = (start + slot) * _K
            return pltpu.make_async_copy(
                ei_hbm.at[1, pl.ds(off, _K)], dstb.at[r], sd[r]
            )

        def gather_copy(r, par):
            return pltpu.make_async_copy(
                x_hbm.at[srcb.at[r]], rows[par], sg[par]
            )

        def scat_copy(r, par):
            return pltpu.make_async_copy(
                rows[par], agg_sh.at[dstb.at[r]], ss[par]
            )

        zero_cp = pltpu.make_async_copy(
            z_hbm.at[pl.ds(0, nr)], agg_sh.at[pl.ds(s * nr, nr)], sz
        )
        zero_cp.start()
        if tail:
            @pl.when(s == _NUM_SUBCORES - 1)
            def _():
                pltpu.async_copy(
                    z_hbm.at[pl.ds(0, tail)],
                    agg_sh.at[pl.ds(nr * _NUM_SUBCORES, tail)],
                    sz,
                )

        src_copy(0, 0).start()
        src_copy(1, 1).start()
        src_copy(2, 2).start()
        dst_copy(0, 0).start()
        dst_copy(1, 1).start()
        src_copy(0, 0).wait()
        gather_copy(0, 0).start()
        zero_cp.wait()
        if tail:
            @pl.when(s == _NUM_SUBCORES - 1)
            def _():
                pltpu.make_async_copy(
                    z_hbm.at[pl.ds(0, tail)],
                    agg_sh.at[pl.ds(nr * _NUM_SUBCORES, tail)],
                    sz,
                ).wait()
        plsc.subcore_barrier()

        @pl.loop(0, iters)
        def _(q):
            for j in range(unroll):
                i = unroll * q + j

                @pl.when(jnp.logical_and(i >= 2, i - 2 < cnt))
                def _():
                    scat_copy((j + 2) % 4, (j + 1) % 3).wait()

                @pl.when(i + 1 < cnt)
                def _():
                    src_copy(i + 1, (j + 1) % 4).wait()
                    gather_copy((j + 1) % 4, (j + 1) % 3).start()

                @pl.when(i < cnt)
                def _():
                    gather_copy(j % 4, j % 3).wait()
                    dst_copy(i, j % 4).wait()
                    scat_copy(j % 4, j % 3).start(add=True)

                @pl.when(i + 3 < cnt)
                def _():
                    src_copy(i + 3, (j + 3) % 4).start()

                @pl.when(i + 2 < cnt)
                def _():
                    dst_copy(i + 2, (j + 2) % 4).start()

        plsc.subcore_barrier()
        pltpu.sync_copy(
            agg_sh.at[pl.ds(s * nr, nr)], out_hbm.at[c, pl.ds(s * nr, nr)]
        )
        if tail:
            @pl.when(s == _NUM_SUBCORES - 1)
            def _():
                pltpu.sync_copy(
                    agg_sh.at[pl.ds(nr * _NUM_SUBCORES, tail)],
                    out_hbm.at[c, pl.ds(nr * _NUM_SUBCORES, tail)],
                )

    return k(x, edge_index, zeros)


def _x_self(x, W_self, b_row):
    N, D = x.shape

    def body(x_ref, ws_ref, b_ref, o_ref):
        o_ref[...] = (
            jnp.dot(x_ref[...], ws_ref[...], preferred_element_type=jnp.float32)
            + b_ref[...]
        )

    return pl.pallas_call(
        body, out_shape=jax.ShapeDtypeStruct((N, D), jnp.float32)
    )(x, W_self, b_row)


def _post(agg2, x_self, W_msg, att_w_row, batch_row, G):
    N, D = x_self.shape

    def body(agg_ref, xs_ref, wm_ref, aw_ref, bat_ref, out_ref):
        aggm = jnp.dot(
            agg_ref[0] + agg_ref[1], wm_ref[...],
            preferred_element_type=jnp.float32,
        )
        h = jnp.maximum(aggm + xs_ref[...], 0.0)
        logits = lax.dot_general(
            aw_ref[...], h, (((1,), (1,)), ((), ())),
            preferred_element_type=jnp.float32,
        )
        bat = bat_ref[...]
        gid = lax.broadcasted_iota(jnp.int32, (G, N), 0)
        mask = gid == bat
        big_neg = jnp.float32(-1e30)
        seg_max = jnp.max(jnp.where(mask, logits, big_neg), axis=1, keepdims=True)
        maxn = jnp.sum(jnp.where(mask, seg_max, 0.0), axis=0, keepdims=True)
        w = jnp.exp(logits - maxn)
        denom = jnp.sum(jnp.where(mask, w, 0.0), axis=1, keepdims=True)
        denn = jnp.sum(jnp.where(mask, denom, 0.0), axis=0, keepdims=True)
        wn = w / denn
        out_ref[...] = jnp.dot(
            jnp.where(mask, wn, 0.0), h, preferred_element_type=jnp.float32
        )

    return pl.pallas_call(
        body, out_shape=jax.ShapeDtypeStruct((G, D), jnp.float32)
    )(agg2, x_self, W_msg, att_w_row, batch_row)


def kernel(x, edge_index, batch, W_msg, W_self, b, att_w, att_b):
    N, D = x.shape
    G = 64
    zeros = jnp.zeros((640, D), jnp.float32)
    agg2 = _edge_agg_sc(x, edge_index, zeros)
    x_self = _x_self(x, W_self, b.reshape(1, D))
    out = _post(agg2, x_self, W_msg, att_w.reshape(1, D),
                batch.reshape(1, N), G)
    return out


# speedup vs baseline: 1.1143x; 1.0823x over previous
import functools

import jax
import jax.numpy as jnp
from jax import lax
from jax.experimental import pallas as pl
from jax.experimental.pallas import tpu as pltpu
from jax.experimental.pallas import tpu_sc as plsc

_NUM_CORES = 2
_NUM_SUBCORES = 16
_K = 128


def _edge_agg_sc(x, edge_index, zeros):
    N, D = x.shape
    E = edge_index.shape[1]
    NW = _NUM_CORES * _NUM_SUBCORES
    chunks = E // _K
    base_cnt = chunks // NW
    extra = chunks - base_cnt * NW
    slots = base_cnt + (1 if extra else 0)
    unroll = 12
    iters = (slots + 2 + unroll) // unroll
    nr = (N // _NUM_SUBCORES) // 8 * 8
    tail = N - nr * _NUM_SUBCORES
    mesh = plsc.VectorSubcoreMesh(core_axis_name="c", subcore_axis_name="s")

    @functools.partial(
        pl.kernel,
        out_type=jax.ShapeDtypeStruct((_NUM_CORES, N, D), jnp.float32),
        mesh=mesh,
        scratch_types=[
            pltpu.VMEM((4, _K), jnp.int32),
            pltpu.VMEM((4, _K), jnp.int32),
            pltpu.VMEM((_K, D), jnp.float32),
            pltpu.VMEM((_K, D), jnp.float32),
            pltpu.VMEM((_K, D), jnp.float32),
            pltpu.VMEM_SHARED((N, D), jnp.float32),
        ] + [pltpu.SemaphoreType.DMA] * 15,
    )
    def k(x_hbm, ei_hbm, z_hbm, out_hbm,
          srcb, dstb, rows0, rows1, rows2, agg_sh,
          sg0, sg1, sg2, ss0, ss1, ss2,
          sa0, sa1, sa2, sa3, sd0, sd1, sd2, sd3, sz):
        c = lax.axis_index("c")
        s = lax.axis_index("s")
        wid = c * _NUM_SUBCORES + s
        start = wid * base_cnt + jnp.minimum(wid, extra)
        cnt = base_cnt + jnp.where(wid < extra, 1, 0)
        rows = (rows0, rows1, rows2)
        sg = (sg0, sg1, sg2)
        ss = (ss0, ss1, ss2)
        sa = (sa0, sa1, sa2, sa3)
        sd = (sd0, sd1, sd2, sd3)

        def src_copy(slot, r):
            off = (start + slot) * _K
            return pltpu.make_async_copy(
                ei_hbm.at[0, pl.ds(off, _K)], srcb.at[r], sa[r]
            )

        def dst_copy(slot, r):
            off = (start + slot) * _K
            return pltpu.make_async_copy(
                ei_hbm.at[1, pl.ds(off, _K)], dstb.at[r], sd[r]
            )

        def gather_copy(r, par):
            return pltpu.make_async_copy(
                x_hbm.at[srcb.at[r]], rows[par], sg[par]
            )

        def scat_copy(r, par):
            return pltpu.make_async_copy(
                rows[par], agg_sh.at[dstb.at[r]], ss[par]
            )

        zero_cp = pltpu.make_async_copy(
            z_hbm.at[pl.ds(0, nr)], agg_sh.at[pl.ds(s * nr, nr)], sz
        )
        zero_cp.start()
        if tail:
            @pl.when(s == _NUM_SUBCORES - 1)
            def _():
                pltpu.async_copy(
                    z_hbm.at[pl.ds(0, tail)],
                    agg_sh.at[pl.ds(nr * _NUM_SUBCORES, tail)],
                    sz,
                )

        src_copy(0, 0).start()
        src_copy(1, 1).start()
        src_copy(2, 2).start()
        src_copy(3, 3).start()
        dst_copy(0, 0).start()
        dst_copy(1, 1).start()
        src_copy(0, 0).wait()
        gather_copy(0, 0).start()
        src_copy(1, 1).wait()
        gather_copy(1, 1).start()
        zero_cp.wait()
        if tail:
            @pl.when(s == _NUM_SUBCORES - 1)
            def _():
                pltpu.make_async_copy(
                    z_hbm.at[pl.ds(0, tail)],
                    agg_sh.at[pl.ds(nr * _NUM_SUBCORES, tail)],
                    sz,
                ).wait()
        plsc.subcore_barrier()

        @pl.loop(0, iters)
        def _(q):
            for j in range(unroll):
                i = unroll * q + j

                @pl.when(jnp.logical_and(i >= 1, i - 1 < cnt))
                def _():
                    scat_copy((j + 3) % 4, (j + 2) % 3).wait()

                @pl.when(i + 2 < cnt)
                def _():
                    src_copy(i + 2, (j + 2) % 4).wait()
                    gather_copy((j + 2) % 4, (j + 2) % 3).start()

                @pl.when(i < cnt)
                def _():
                    gather_copy(j % 4, j % 3).wait()
                    dst_copy(i, j % 4).wait()
                    scat_copy(j % 4, j % 3).start(add=True)

                @pl.when(i + 4 < cnt)
                def _():
                    src_copy(i + 4, j % 4).start()

                @pl.when(i + 2 < cnt)
                def _():
                    dst_copy(i + 2, (j + 2) % 4).start()

        plsc.subcore_barrier()
        pltpu.sync_copy(
            agg_sh.at[pl.ds(s * nr, nr)], out_hbm.at[c, pl.ds(s * nr, nr)]
        )
        if tail:
            @pl.when(s == _NUM_SUBCORES - 1)
            def _():
                pltpu.sync_copy(
                    agg_sh.at[pl.ds(nr * _NUM_SUBCORES, tail)],
                    out_hbm.at[c, pl.ds(nr * _NUM_SUBCORES, tail)],
                )

    return k(x, edge_index, zeros)


def _x_self(x, W_self, b_row):
    N, D = x.shape

    def body(x_ref, ws_ref, b_ref, o_ref):
        o_ref[...] = (
            jnp.dot(x_ref[...], ws_ref[...], preferred_element_type=jnp.float32)
            + b_ref[...]
        )

    return pl.pallas_call(
        body, out_shape=jax.ShapeDtypeStruct((N, D), jnp.float32)
    )(x, W_self, b_row)


def _post(agg2, x_self, W_msg, att_w_row, batch_row, G):
    N, D = x_self.shape

    def body(agg_ref, xs_ref, wm_ref, aw_ref, bat_ref, out_ref):
        aggm = jnp.dot(
            agg_ref[0] + agg_ref[1], wm_ref[...],
            preferred_element_type=jnp.float32,
        )
        h = jnp.maximum(aggm + xs_ref[...], 0.0)
        logits = lax.dot_general(
            aw_ref[...], h, (((1,), (1,)), ((), ())),
            preferred_element_type=jnp.float32,
        )
        bat = bat_ref[...]
        gid = lax.broadcasted_iota(jnp.int32, (G, N), 0)
        mask = gid == bat
        big_neg = jnp.float32(-1e30)
        seg_max = jnp.max(jnp.where(mask, logits, big_neg), axis=1, keepdims=True)
        maxn = jnp.sum(jnp.where(mask, seg_max, 0.0), axis=0, keepdims=True)
        w = jnp.exp(logits - maxn)
        denom = jnp.sum(jnp.where(mask, w, 0.0), axis=1, keepdims=True)
        denn = jnp.sum(jnp.where(mask, denom, 0.0), axis=0, keepdims=True)
        wn = w / denn
        out_ref[...] = jnp.dot(
            jnp.where(mask, wn, 0.0), h, preferred_element_type=jnp.float32
        )

    return pl.pallas_call(
        body, out_shape=jax.ShapeDtypeStruct((G, D), jnp.float32)
    )(agg2, x_self, W_msg, att_w_row, batch_row)


def kernel(x, edge_index, batch, W_msg, W_self, b, att_w, att_b):
    N, D = x.shape
    G = 64
    zeros = jnp.zeros((640, D), jnp.float32)
    agg2 = _edge_agg_sc(x, edge_index, zeros)
    x_self = _x_self(x, W_self, b.reshape(1, D))
    out = _post(agg2, x_self, W_msg, att_w.reshape(1, D),
                batch.reshape(1, N), G)
    return out
